# baseline (speedup 1.0000x reference)
import functools

import jax
import jax.numpy as jnp
from jax import lax
from jax.experimental import pallas as pl
from jax.experimental.pallas import tpu as pltpu
from jax.experimental.pallas import tpu_sc as plsc

TM = 32


def _router_body(T, E, RPAD, NT, x_ref, gw_ref,
                 dest_ref, xs_ref, rwt_ref, te_ref, act_ref):
    x = x_ref[...].reshape(T, x_ref.shape[-1])
    gw = gw_ref[...]
    logits = lax.dot_general(x, gw, (((1,), (1,)), ((), ())),
                             preferred_element_type=jnp.float32)
    m = jnp.max(logits, axis=1, keepdims=True)
    p = 1.0 / jnp.sum(jnp.exp(logits - m), axis=1, keepdims=True)
    wt = p / (p + 1e-9)

    e_iota = lax.broadcasted_iota(jnp.int32, (T, E), 1)
    eid = jnp.min(jnp.where(logits >= m, e_iota, E), axis=1, keepdims=True)
    onehot = (e_iota == eid).astype(jnp.float32)

    counts = lax.dot_general(onehot, jnp.ones((T, 1), jnp.float32),
                             (((0,), (0,)), ((), ())),
                             preferred_element_type=jnp.float32)
    padded = (((counts.astype(jnp.int32) + (TM - 1)) // TM)
              * TM).astype(jnp.float32)

    r_e = lax.broadcasted_iota(jnp.int32, (E, E), 0)
    c_e = lax.broadcasted_iota(jnp.int32, (E, E), 1)
    tri_e = (c_e < r_e).astype(jnp.float32)
    p_ex = lax.dot_general(tri_e, padded, (((1,), (0,)), ((), ())),
                           precision=lax.Precision.HIGHEST,
                           preferred_element_type=jnp.float32)
    p_end = p_ex + padded
    total = jnp.sum(padded)

    r_t = lax.broadcasted_iota(jnp.int32, (T, T), 0)
    c_t = lax.broadcasted_iota(jnp.int32, (T, T), 1)
    tri_t = (c_t < r_t).astype(jnp.float32)
    posm = lax.dot_general(tri_t, onehot, (((1,), (0,)), ((), ())),
                           preferred_element_type=jnp.float32)
    pos = jnp.sum(posm * onehot, axis=1, keepdims=True)
    base = lax.dot_general(onehot, p_ex, (((1,), (0,)), ((), ())),
                           preferred_element_type=jnp.float32)
    dest = (base + pos).astype(jnp.int32)
    ident = (r_t == c_t).astype(jnp.float32)
    d_hi = dest // TM
    d_cols = jnp.concatenate(
        [d_hi.astype(jnp.float32),
         (dest - d_hi * TM).astype(jnp.float32)], axis=1)
    d_rows = lax.dot_general(d_cols, ident, (((0,), (0,)), ((), ())),
                             preferred_element_type=jnp.float32)
    dest_ref[...] = (float(TM) * d_rows[0:1, :]
                     + d_rows[1:2, :]).astype(jnp.int32)

    r_cols = lax.broadcasted_iota(jnp.int32, (T, RPAD), 1)
    mt = (r_cols == dest).astype(jnp.float32)
    xs_ref[...] = lax.dot_general(mt, x, (((0,), (0,)), ((), ())),
                                  preferred_element_type=jnp.float32
                                  ).astype(jnp.bfloat16)
    wt_a = wt.astype(jnp.bfloat16).astype(jnp.float32)
    wt_cols = jnp.concatenate([wt_a, wt - wt_a], axis=1)
    rwt2 = lax.dot_general(mt, wt_cols, (((0,), (0,)), ((), ())),
                           preferred_element_type=jnp.float32)
    rwt_ref[...] = jnp.sum(rwt2, axis=1, keepdims=True)

    i_tile = lax.broadcasted_iota(jnp.int32, (1, NT), 1).astype(jnp.float32) * TM
    r_eff = jnp.minimum(i_tile, total - TM)
    te_ref[...] = jnp.sum((p_end <= r_eff).astype(jnp.float32),
                          axis=0, keepdims=True).astype(jnp.int32)
    act_ref[...] = (i_tile < total).astype(jnp.int32)


def _ffn_body(te_ref, act_ref, xs_ref, w1_ref, b1_ref, w2_ref, b2_ref,
              wt_ref, o_ref):
    i = pl.program_id(0)

    @pl.when(act_ref[i] == 1)
    def _():
        e = te_ref[i]
        sub = e - 2 * (e // 2)
        x = xs_ref[...].astype(jnp.float32)
        w1b = w1_ref[pl.ds(sub, 1), :, :][0]
        w2b = w2_ref[pl.ds(sub, 1), :, :][0]
        h = (lax.dot_general(x, w1b, (((1,), (0,)), ((), ())),
                             preferred_element_type=jnp.float32)
             + b1_ref[pl.ds(e, 1), :])
        h = jnp.maximum(h, 0.0)
        o = (lax.dot_general(h, w2b, (((1,), (0,)), ((), ())),
                             preferred_element_type=jnp.float32)
             + b2_ref[pl.ds(e, 1), :])
        o_ref[...] = o * wt_ref[...]


def _sc_gather(table, idx, out_rows):
    rows, d = out_rows, table.shape[1]
    info = plsc.get_sparse_core_info()
    nw = info.num_cores * info.num_subcores
    b_per_w = rows // nw
    mesh = plsc.VectorSubcoreMesh(core_axis_name="c", subcore_axis_name="s")

    @functools.partial(
        pl.kernel, mesh=mesh,
        out_type=jax.ShapeDtypeStruct((rows, d), jnp.float32),
        scratch_types=[
            pltpu.VMEM((b_per_w,), jnp.int32),
            pltpu.VMEM((b_per_w, d), jnp.float32),
            pltpu.SemaphoreType.DMA,
        ],
    )
    def gather_k(table_hbm, idx_hbm, out_hbm, idx_v, rows_v, sem):
        wid = lax.axis_index("s") * info.num_cores + lax.axis_index("c")
        base = wid * b_per_w
        pltpu.sync_copy(idx_hbm.at[pl.ds(base, b_per_w)], idx_v)
        pltpu.async_copy(table_hbm.at[idx_v], rows_v, sem).wait()
        pltpu.sync_copy(rows_v, out_hbm.at[pl.ds(base, b_per_w)])

    return gather_k(table, idx)


def kernel(x, gate_w, w1, b1, w2, b2):
    orig_shape = x.shape
    D = x.shape[-1]
    T = x.size // D
    E, H = w1.shape[0], w1.shape[2]
    RPAD = -(-(T + E * (TM - 1)) // 256) * 256
    NT = RPAD // TM

    dest, xs, rwt, te, act = pl.pallas_call(
        functools.partial(_router_body, T, E, RPAD, NT),
        out_shape=[
            jax.ShapeDtypeStruct((1, T), jnp.int32),
            jax.ShapeDtypeStruct((RPAD, D), jnp.bfloat16),
            jax.ShapeDtypeStruct((RPAD, 1), jnp.float32),
            jax.ShapeDtypeStruct((1, NT), jnp.int32),
            jax.ShapeDtypeStruct((1, NT), jnp.int32),
        ],
    )(x, gate_w)

    grid_spec = pltpu.PrefetchScalarGridSpec(
        num_scalar_prefetch=2,
        grid=(NT,),
        in_specs=[
            pl.BlockSpec((TM, D), lambda i, te, act: (i, 0)),
            pl.BlockSpec((2, D, H), lambda i, te, act: (te[i] // 2, 0, 0)),
            pl.BlockSpec((E, H), lambda i, te, act: (0, 0)),
            pl.BlockSpec((2, H, D), lambda i, te, act: (te[i] // 2, 0, 0)),
            pl.BlockSpec((E, D), lambda i, te, act: (0, 0)),
            pl.BlockSpec((TM, 1), lambda i, te, act: (i, 0)),
        ],
        out_specs=pl.BlockSpec((TM, D), lambda i, te, act: (i, 0)),
    )
    osort = pl.pallas_call(
        _ffn_body,
        grid_spec=grid_spec,
        out_shape=jax.ShapeDtypeStruct((RPAD, D), jnp.float32),
    )(te.reshape(NT), act.reshape(NT), xs, w1, b1, w2, b2, rwt)

    out = _sc_gather(osort, dest.reshape(T), T)
    return out.reshape(orig_shape)


# speedup vs baseline: 1.3014x; 1.3014x over previous
import functools

import jax
import jax.numpy as jnp
from jax import lax
from jax.experimental import pallas as pl
from jax.experimental.pallas import tpu as pltpu
from jax.experimental.pallas import tpu_sc as plsc

TM = 32


def _router_body(T, E, RPAD, NT, x_ref, gw_ref,
                 dest_ref, xs_ref, rwt_ref, te_ref, act_ref):
    x = x_ref[...].reshape(T, x_ref.shape[-1])
    gw = gw_ref[...]
    logits = lax.dot_general(x, gw, (((1,), (1,)), ((), ())),
                             preferred_element_type=jnp.float32)
    m = jnp.max(logits, axis=1, keepdims=True)
    p = 1.0 / jnp.sum(jnp.exp(logits - m), axis=1, keepdims=True)
    wt = p / (p + 1e-9)

    e_iota = lax.broadcasted_iota(jnp.int32, (T, E), 1)
    eid = jnp.min(jnp.where(logits >= m, e_iota, E), axis=1, keepdims=True)
    onehot = (e_iota == eid).astype(jnp.float32)

    counts = lax.dot_general(onehot, jnp.ones((T, 1), jnp.float32),
                             (((0,), (0,)), ((), ())),
                             preferred_element_type=jnp.float32)
    padded = (((counts.astype(jnp.int32) + (TM - 1)) // TM)
              * TM).astype(jnp.float32)

    r_e = lax.broadcasted_iota(jnp.int32, (E, E), 0)
    c_e = lax.broadcasted_iota(jnp.int32, (E, E), 1)
    tri_e = (c_e < r_e).astype(jnp.float32)
    p_ex = lax.dot_general(tri_e, padded, (((1,), (0,)), ((), ())),
                           precision=lax.Precision.HIGHEST,
                           preferred_element_type=jnp.float32)
    p_end = p_ex + padded
    total = jnp.sum(padded)

    r_t = lax.broadcasted_iota(jnp.int32, (T, T), 0)
    c_t = lax.broadcasted_iota(jnp.int32, (T, T), 1)
    tri_t = (c_t < r_t).astype(jnp.float32)
    posm = lax.dot_general(tri_t, onehot, (((1,), (0,)), ((), ())),
                           preferred_element_type=jnp.float32)
    pos = jnp.sum(posm * onehot, axis=1, keepdims=True)
    base = lax.dot_general(onehot, p_ex, (((1,), (0,)), ((), ())),
                           preferred_element_type=jnp.float32)
    dest = (base + pos).astype(jnp.int32)
    ident = (r_t == c_t).astype(jnp.float32)
    d_hi = dest // TM
    d_cols = jnp.concatenate(
        [d_hi.astype(jnp.float32),
         (dest - d_hi * TM).astype(jnp.float32)], axis=1)
    d_rows = lax.dot_general(d_cols, ident, (((0,), (0,)), ((), ())),
                             preferred_element_type=jnp.float32)
    dest_ref[...] = (float(TM) * d_rows[0:1, :]
                     + d_rows[1:2, :]).astype(jnp.int32)

    r_cols = lax.broadcasted_iota(jnp.int32, (T, RPAD), 1)
    mt = (r_cols == dest).astype(jnp.float32)
    xs_ref[...] = lax.dot_general(mt, x, (((0,), (0,)), ((), ())),
                                  preferred_element_type=jnp.float32
                                  ).astype(jnp.bfloat16)
    wt_a = wt.astype(jnp.bfloat16).astype(jnp.float32)
    wt_cols = jnp.concatenate([wt_a, wt - wt_a], axis=1)
    rwt2 = lax.dot_general(mt, wt_cols, (((0,), (0,)), ((), ())),
                           preferred_element_type=jnp.float32)
    rwt_ref[...] = jnp.sum(rwt2, axis=1, keepdims=True)

    i_tile = lax.broadcasted_iota(jnp.int32, (1, NT), 1).astype(jnp.float32) * TM
    r_eff = jnp.minimum(i_tile, total - TM)
    te_ref[...] = jnp.sum((p_end <= r_eff).astype(jnp.float32),
                          axis=0, keepdims=True).astype(jnp.int32)
    act_ref[...] = (i_tile < total).astype(jnp.int32)


def _ffn_body(te_ref, act_ref, xs_ref, w1_ref, b1_ref, w2_ref, b2_ref,
              wt_ref, o_ref):
    i = pl.program_id(0)

    @pl.when(act_ref[i] == 1)
    def _():
        e = te_ref[i]
        x = xs_ref[...].astype(jnp.float32)
        h = (lax.dot_general(x, w1_ref[0], (((1,), (0,)), ((), ())),
                             preferred_element_type=jnp.float32)
             + b1_ref[pl.ds(e, 1), :])
        h = jnp.maximum(h, 0.0)
        o = (lax.dot_general(h, w2_ref[0], (((1,), (0,)), ((), ())),
                             preferred_element_type=jnp.float32)
             + b2_ref[pl.ds(e, 1), :])
        o_ref[...] = o * wt_ref[...]


def _sc_gather(table, idx, out_rows):
    rows, d = out_rows, table.shape[1]
    info = plsc.get_sparse_core_info()
    nw = info.num_cores * info.num_subcores
    b_per_w = rows // nw
    mesh = plsc.VectorSubcoreMesh(core_axis_name="c", subcore_axis_name="s")

    @functools.partial(
        pl.kernel, mesh=mesh,
        out_type=jax.ShapeDtypeStruct((rows, d), jnp.float32),
        scratch_types=[
            pltpu.VMEM((b_per_w,), jnp.int32),
            pltpu.VMEM((b_per_w, d), jnp.float32),
            pltpu.SemaphoreType.DMA,
        ],
    )
    def gather_k(table_hbm, idx_hbm, out_hbm, idx_v, rows_v, sem):
        wid = lax.axis_index("s") * info.num_cores + lax.axis_index("c")
        base = wid * b_per_w
        pltpu.sync_copy(idx_hbm.at[pl.ds(base, b_per_w)], idx_v)
        pltpu.async_copy(table_hbm.at[idx_v], rows_v, sem).wait()
        pltpu.sync_copy(rows_v, out_hbm.at[pl.ds(base, b_per_w)])

    return gather_k(table, idx)


def kernel(x, gate_w, w1, b1, w2, b2):
    orig_shape = x.shape
    D = x.shape[-1]
    T = x.size // D
    E, H = w1.shape[0], w1.shape[2]
    RPAD = -(-(T + E * (TM - 1)) // 256) * 256
    NT = RPAD // TM

    dest, xs, rwt, te, act = pl.pallas_call(
        functools.partial(_router_body, T, E, RPAD, NT),
        out_shape=[
            jax.ShapeDtypeStruct((1, T), jnp.int32),
            jax.ShapeDtypeStruct((RPAD, D), jnp.bfloat16),
            jax.ShapeDtypeStruct((RPAD, 1), jnp.float32),
            jax.ShapeDtypeStruct((1, NT), jnp.int32),
            jax.ShapeDtypeStruct((1, NT), jnp.int32),
        ],
    )(x, gate_w)

    grid_spec = pltpu.PrefetchScalarGridSpec(
        num_scalar_prefetch=2,
        grid=(NT,),
        in_specs=[
            pl.BlockSpec((TM, D), lambda i, te, act: (i, 0)),
            pl.BlockSpec((1, D, H), lambda i, te, act: (te[i], 0, 0)),
            pl.BlockSpec((E, H), lambda i, te, act: (0, 0)),
            pl.BlockSpec((1, H, D), lambda i, te, act: (te[i], 0, 0)),
            pl.BlockSpec((E, D), lambda i, te, act: (0, 0)),
            pl.BlockSpec((TM, 1), lambda i, te, act: (i, 0)),
        ],
        out_specs=pl.BlockSpec((TM, D), lambda i, te, act: (i, 0)),
    )
    osort = pl.pallas_call(
        _ffn_body,
        grid_spec=grid_spec,
        out_shape=jax.ShapeDtypeStruct((RPAD, D), jnp.float32),
    )(te.reshape(NT), act.reshape(NT), xs, w1, b1, w2, b2, rwt)

    out = _sc_gather(osort, dest.reshape(T), T)
    return out.reshape(orig_shape)
